# baseline (speedup 1.0000x reference)
import functools

import jax
import jax.numpy as jnp
from jax import lax
from jax.experimental import pallas as pl
from jax.experimental.pallas import tpu as pltpu
from jax.experimental.pallas import tpu_sc as plsc

N = 8192
NSEG = 8
K = 20

_pcall = pl.pallas_call

_G = 8
_C = N // _G
_MASKVAL = 0x7F7F8000
_BIGF = float(3.4028235e38)
_LOCMASK = 0x3FF
_KEYMASK = -1024


def _knn_uv_body(xT_ref, x_ref, brow_ref, bcol_ref, wu_ref, bu_ref, wv_ref,
                 idx_ref, u_ref, v_ref):
    x = x_ref[...]
    xT = xT_ref[...]
    u_ref[...] = jnp.dot(x, wu_ref[...], preferred_element_type=jnp.float32) + bu_ref[...]
    v_ref[...] = jnp.dot(x, wv_ref[...], preferred_element_type=jnp.float32)

    x2c = jnp.sum(xT * xT, axis=0, keepdims=True)
    x2r = jnp.sum(x * x, axis=1, keepdims=True)
    brow = brow_ref[...]
    r = x.shape[0]
    loc = lax.broadcasted_iota(jnp.int32, (r, _C), 1)
    maskf = lax.bitcast_convert_type(_MASKVAL | loc, jnp.float32)
    b_lo = brow_ref[0, 0]
    b_hi = brow_ref[r - 1, 0]

    def _chunk_topk(sl):
        distg = x2r + x2c[:, sl] - 2.0 * jnp.dot(
            x, xT[:, sl], preferred_element_type=jnp.float32)
        dc = jnp.maximum(distg, 1e-30)
        pk = lax.bitcast_convert_type(
            (lax.bitcast_convert_type(dc, jnp.int32) & _KEYMASK) | loc,
            jnp.float32)
        pk = jnp.where(brow != bcol_ref[:, sl], maskf, pk)
        thr = jnp.full((r, 1), -1.0, jnp.float32)
        picks = []
        for _ in range(K):
            cand = jnp.where(pk > thr, pk, _BIGF)
            m = jnp.min(cand, axis=1)
            picks.append(m)
            thr = m[:, None]
        return jnp.stack(picks, axis=1)

    vals = []
    for g in range(_G):
        sl = slice(g * _C, (g + 1) * _C)
        c_lo = bcol_ref[0, g * _C]
        c_hi = bcol_ref[0, (g + 1) * _C - 1]
        pred = jnp.logical_and(b_lo <= c_hi, c_lo <= b_hi)
        vals.append(lax.cond(
            pred,
            functools.partial(_chunk_topk, sl),
            lambda: jnp.full((r, K), _BIGF, jnp.float32)))
    V = jnp.concatenate(vals, axis=1)

    iot = lax.broadcasted_iota(jnp.int32, V.shape, 1).astype(jnp.float32)
    alive = jnp.ones(V.shape, jnp.bool_)
    idxs = []
    for _ in range(K):
        cand = jnp.where(alive, V, _BIGF)
        m = jnp.min(cand, axis=1)[:, None]
        pos = jnp.min(jnp.where(cand == m, iot, _BIGF), axis=1)[:, None]
        pos_i = pos.astype(jnp.int32)
        mloc = lax.bitcast_convert_type(m, jnp.int32) & _LOCMASK
        idxs.append((pos_i // K) * _C + mloc)
        alive = alive & (iot != pos)
    idx_ref[...] = jnp.concatenate(idxs, axis=1)


@functools.lru_cache(maxsize=None)
def _make_knn_uv(d, dh, r):
    return _pcall(
        _knn_uv_body,
        grid=(N // r,),
        in_specs=[
            pl.BlockSpec((d, N), lambda i: (0, 0)),
            pl.BlockSpec((r, d), lambda i: (i, 0)),
            pl.BlockSpec((r, 1), lambda i: (i, 0)),
            pl.BlockSpec((1, N), lambda i: (0, 0)),
            pl.BlockSpec((d, dh), lambda i: (0, 0)),
            pl.BlockSpec((1, dh), lambda i: (0, 0)),
            pl.BlockSpec((d, dh), lambda i: (0, 0)),
        ],
        out_specs=[
            pl.BlockSpec((r, K), lambda i: (i, 0)),
            pl.BlockSpec((r, dh), lambda i: (i, 0)),
            pl.BlockSpec((r, dh), lambda i: (i, 0)),
        ],
        out_shape=[
            jax.ShapeDtypeStruct((N, K), jnp.int32),
            jax.ShapeDtypeStruct((N, dh), jnp.float32),
            jax.ShapeDtypeStruct((N, dh), jnp.float32),
        ],
    )


def _edge_body(g_ref, u_ref, w2_ref, b2_ref, out_ref):
    u = u_ref[...]
    w2 = w2_ref[...]
    acc = None
    for k in range(K):
        h = jnp.maximum(u + g_ref[k], 0.0)
        hw = jnp.dot(h, w2, preferred_element_type=jnp.float32)
        acc = hw if acc is None else jnp.maximum(acc, hw)
    out_ref[...] = acc + b2_ref[...]


@functools.lru_cache(maxsize=None)
def _make_edge(dg, dh, r):
    return _pcall(
        _edge_body,
        grid=(N // r,),
        in_specs=[
            pl.BlockSpec((K, r, dg), lambda i: (0, i, 0)),
            pl.BlockSpec((r, dg), lambda i: (i, 0)),
            pl.BlockSpec((dg, dh), lambda i: (0, 0)),
            pl.BlockSpec((1, dh), lambda i: (0, 0)),
        ],
        out_specs=pl.BlockSpec((r, dh), lambda i: (i, 0)),
        out_shape=jax.ShapeDtypeStruct((N, dh), jnp.float32),
    )


@functools.lru_cache(maxsize=None)
def _make_sc_gather(dh):
    info = plsc.get_sparse_core_info()
    nc, ns = info.num_cores, info.num_subcores
    nw = nc * ns
    tot = N * K
    per_w = tot // nw
    ch = 128
    nch = per_w // ch
    mesh = plsc.VectorSubcoreMesh(core_axis_name="c", subcore_axis_name="s")

    @functools.partial(
        pl.kernel,
        mesh=mesh,
        out_type=jax.ShapeDtypeStruct((tot, dh), jnp.float32),
        scratch_types=[
            pltpu.VMEM((ch,), jnp.int32),
            pltpu.VMEM((ch,), jnp.int32),
            pltpu.VMEM((ch, dh), jnp.float32),
            pltpu.VMEM((ch, dh), jnp.float32),
            pltpu.SemaphoreType.DMA,
            pltpu.SemaphoreType.DMA,
            pltpu.SemaphoreType.DMA,
            pltpu.SemaphoreType.DMA,
        ],
    )
    def gather(table_hbm, idx_hbm, out_hbm, idx0, idx1, rows0, rows1,
               sg0, sg1, sw0, sw1):
        wid = lax.axis_index("s") * nc + lax.axis_index("c")
        base = wid * per_w

        def body(j, carry):
            for p, (idx_v, rows_v, sg, sw) in enumerate(
                    ((idx0, rows0, sg0, sw0), (idx1, rows1, sg1, sw1))):
                off = base + (2 * j + p) * ch
                pltpu.sync_copy(idx_hbm.at[pl.ds(off, ch)], idx_v)

                @pl.when(j > 0)
                def _():
                    pltpu.make_async_copy(
                        rows_v, out_hbm.at[pl.ds(off, ch)], sw).wait()
                pltpu.make_async_copy(table_hbm.at[idx_v], rows_v, sg).start()
            for p, (idx_v, rows_v, sg, sw) in enumerate(
                    ((idx0, rows0, sg0, sw0), (idx1, rows1, sg1, sw1))):
                off = base + (2 * j + p) * ch
                pltpu.make_async_copy(table_hbm.at[idx_v], rows_v, sg).wait()
                pltpu.make_async_copy(
                    rows_v, out_hbm.at[pl.ds(off, ch)], sw).start()
            return carry

        lax.fori_loop(0, nch // 2, body, 0)
        last = base + (nch - 2) * ch
        pltpu.make_async_copy(rows0, out_hbm.at[pl.ds(last, ch)], sw0).wait()
        pltpu.make_async_copy(
            rows1, out_hbm.at[pl.ds(last + ch, ch)], sw1).wait()

    return gather


def _gather_rows(v, idx_flat, dh):
    return _make_sc_gather(dh)(v, idx_flat)


def _pool_body(x_ref, brow_ref, out_ref):
    @pl.when(pl.program_id(0) == 0)
    def _():
        out_ref[...] = jnp.full_like(out_ref, -jnp.inf)

    x = x_ref[...]
    b = brow_ref[...]
    rows = []
    for s in range(NSEG):
        rows.append(jnp.max(jnp.where(b == s, x, -jnp.inf), axis=0))
    out_ref[...] = jnp.maximum(out_ref[...], jnp.stack(rows, axis=0))


@functools.lru_cache(maxsize=None)
def _make_pool(f, r):
    return _pcall(
        _pool_body,
        grid=(N // r,),
        in_specs=[
            pl.BlockSpec((r, f), lambda i: (i, 0)),
            pl.BlockSpec((r, 1), lambda i: (i, 0)),
        ],
        out_specs=pl.BlockSpec((NSEG, f), lambda i: (0, 0)),
        out_shape=jax.ShapeDtypeStruct((NSEG, f), jnp.float32),
    )


def _head_body(p_ref, w1_ref, b1_ref, gam_ref, bet_ref, w2_ref, b2_ref, out_ref):
    h = jnp.dot(p_ref[...], w1_ref[...], preferred_element_type=jnp.float32) + b1_ref[...]
    inv = jnp.float32(1.0) / jnp.sqrt(jnp.float32(1.0 + 1e-5))
    h = h * (gam_ref[...] * inv) + bet_ref[...]
    h = jnp.maximum(h, 0.0)
    o = jnp.dot(h, w2_ref[...], preferred_element_type=jnp.float32) + b2_ref[...]
    m = jnp.max(o, axis=1, keepdims=True)
    ls = jnp.log(jnp.sum(jnp.exp(o - m), axis=1, keepdims=True))
    out_ref[...] = o - m - ls


def _make_head(fin, fmid, fout):
    return _pcall(
        _head_body,
        in_specs=[
            pl.BlockSpec((NSEG, fin), lambda: (0, 0)),
            pl.BlockSpec((fin, fmid), lambda: (0, 0)),
            pl.BlockSpec((1, fmid), lambda: (0, 0)),
            pl.BlockSpec((1, fmid), lambda: (0, 0)),
            pl.BlockSpec((1, fmid), lambda: (0, 0)),
            pl.BlockSpec((fmid, fout), lambda: (0, 0)),
            pl.BlockSpec((1, fout), lambda: (0, 0)),
        ],
        out_specs=pl.BlockSpec((NSEG, fout), lambda: (0, 0)),
        out_shape=jax.ShapeDtypeStruct((NSEG, fout), jnp.float32),
    )


def _edge_conv(x, brow, bcol, W1, b1, W2, b2, din_real, r_knn, r_edge):
    d = x.shape[1]
    dh = W2.shape[0]
    dg = max(dh, 128)
    Wv = W1[din_real:, :]
    Wu = W1[:din_real, :] - Wv
    if d != din_real:
        pad = ((0, d - din_real), (0, 0))
        Wu = jnp.pad(Wu, pad)
        Wv = jnp.pad(Wv, pad)
    b1p = b1.reshape(1, -1)
    W2p = W2
    if dg != dh:
        cpad = ((0, 0), (0, dg - dh))
        Wu = jnp.pad(Wu, cpad)
        Wv = jnp.pad(Wv, cpad)
        b1p = jnp.pad(b1p, cpad)
        W2p = jnp.pad(W2, ((0, dg - dh), (0, 0)))
    idx, u, v = _make_knn_uv(d, dg, r_knn)(
        x.T, x, brow, bcol, Wu, b1p, Wv)
    idx_flat = idx.T.reshape(-1)
    g = _gather_rows(v, idx_flat, dg).reshape(K, N, dg)
    return _make_edge(dg, dh, r_edge)(g, u, W2p, b2.reshape(1, -1))


def kernel(pos, params, batch):
    p = params
    brow = batch.reshape(N, 1).astype(jnp.int32)
    bcol = batch.reshape(1, N).astype(jnp.int32)

    x0 = jnp.pad(pos, ((0, 0), (0, 5)))
    x1 = _edge_conv(x0, brow, bcol, p['c1_W1'], p['c1_b1'], p['c1_W2'], p['c1_b2'],
                    3, 512, 256)
    x2 = _edge_conv(x1, brow, bcol, p['c2_W1'], p['c2_b1'], p['c2_W2'], p['c2_b2'],
                    64, 512, 256)
    x3 = _edge_conv(x2, brow, bcol, p['c3_W1'], p['c3_b1'], p['c3_W2'], p['c3_b2'],
                    64, 512, 256)
    x4 = _edge_conv(x3, brow, bcol, p['c4_W1'], p['c4_b1'], p['c4_W2'], p['c4_b2'],
                    128, 512, 256)

    xc = jnp.concatenate([x1, x2, x3, x4], axis=1)
    pooled = _make_pool(512, 256)(xc, brow)
    return _make_head(512, 1024, 40)(
        pooled, p['lin1_W'], p['lin1_b'].reshape(1, -1),
        p['bn1_gamma'].reshape(1, -1), p['bn1_beta'].reshape(1, -1),
        p['lin2_W'], p['lin2_b'].reshape(1, -1))


# speedup vs baseline: 1.7900x; 1.7900x over previous
import functools

import jax
import jax.numpy as jnp
from jax import lax
from jax.experimental import pallas as pl
from jax.experimental.pallas import tpu as pltpu
from jax.experimental.pallas import tpu_sc as plsc

N = 8192
NSEG = 8
K = 20

_pcall = pl.pallas_call

_G = 8
_C = N // _G
_MASKVAL = 0x7F7F8000
_BIGF = float(3.4028235e38)
_LOCMASK = 0x3FF
_KEYMASK = -1024


def _knn_uv_body(xT_ref, x_ref, brow_ref, bcol_ref, wu_ref, bu_ref, wv_ref,
                 idx_ref, u_ref, v_ref):
    x = x_ref[...]
    xT = xT_ref[...]
    u_ref[...] = jnp.dot(x, wu_ref[...], preferred_element_type=jnp.float32) + bu_ref[...]
    v_ref[...] = jnp.dot(x, wv_ref[...], preferred_element_type=jnp.float32)

    x2c = jnp.sum(xT * xT, axis=0, keepdims=True)
    x2r = jnp.sum(x * x, axis=1, keepdims=True)
    brow = brow_ref[...]
    r = x.shape[0]
    loc = lax.broadcasted_iota(jnp.int32, (r, _C), 1)
    maskf = lax.bitcast_convert_type(_MASKVAL | loc, jnp.float32)
    b_lo = brow_ref[0, 0]
    b_hi = brow_ref[r - 1, 0]

    def _chunk_topk(sl):
        distg = x2r + x2c[:, sl] - 2.0 * jnp.dot(
            x, xT[:, sl], preferred_element_type=jnp.float32)
        dc = jnp.maximum(distg, 1e-30)
        pk = lax.bitcast_convert_type(
            (lax.bitcast_convert_type(dc, jnp.int32) & _KEYMASK) | loc,
            jnp.float32)
        pk = jnp.where(brow != bcol_ref[:, sl], maskf, pk)
        thr = jnp.full((r, 1), -1.0, jnp.float32)
        picks = []
        for _ in range(K):
            cand = jnp.where(pk > thr, pk, _BIGF)
            m = jnp.min(cand, axis=1)
            picks.append(m)
            thr = m[:, None]
        return jnp.stack(picks, axis=1)

    vals = []
    for g in range(_G):
        sl = slice(g * _C, (g + 1) * _C)
        c_lo = bcol_ref[0, g * _C]
        c_hi = bcol_ref[0, (g + 1) * _C - 1]
        pred = jnp.logical_and(b_lo <= c_hi, c_lo <= b_hi)
        vals.append(lax.cond(
            pred,
            functools.partial(_chunk_topk, sl),
            lambda: jnp.full((r, K), _BIGF, jnp.float32)))
    V = jnp.concatenate(vals, axis=1)

    iot = lax.broadcasted_iota(jnp.int32, V.shape, 1).astype(jnp.float32)
    alive = jnp.ones(V.shape, jnp.bool_)
    idxs = []
    for _ in range(K):
        cand = jnp.where(alive, V, _BIGF)
        m = jnp.min(cand, axis=1)[:, None]
        pos = jnp.min(jnp.where(cand == m, iot, _BIGF), axis=1)[:, None]
        pos_i = pos.astype(jnp.int32)
        mloc = lax.bitcast_convert_type(m, jnp.int32) & _LOCMASK
        idxs.append((pos_i // K) * _C + mloc)
        alive = alive & (iot != pos)
    idx_ref[...] = jnp.concatenate(idxs, axis=1)


@functools.lru_cache(maxsize=None)
def _make_knn_uv(d, dh, r):
    return _pcall(
        _knn_uv_body,
        grid=(N // r,),
        in_specs=[
            pl.BlockSpec((d, N), lambda i: (0, 0)),
            pl.BlockSpec((r, d), lambda i: (i, 0)),
            pl.BlockSpec((r, 1), lambda i: (i, 0)),
            pl.BlockSpec((1, N), lambda i: (0, 0)),
            pl.BlockSpec((d, dh), lambda i: (0, 0)),
            pl.BlockSpec((1, dh), lambda i: (0, 0)),
            pl.BlockSpec((d, dh), lambda i: (0, 0)),
        ],
        out_specs=[
            pl.BlockSpec((r, K), lambda i: (i, 0)),
            pl.BlockSpec((r, dh), lambda i: (i, 0)),
            pl.BlockSpec((r, dh), lambda i: (i, 0)),
        ],
        out_shape=[
            jax.ShapeDtypeStruct((N, K), jnp.int32),
            jax.ShapeDtypeStruct((N, dh), jnp.float32),
            jax.ShapeDtypeStruct((N, dh), jnp.float32),
        ],
    )


def _edge_body(g_ref, u_ref, w2_ref, b2_ref, out_ref):
    u = u_ref[...]
    w2 = w2_ref[...]
    acc = None
    for k in range(K):
        h = jnp.maximum(u + g_ref[k], 0.0)
        hw = jnp.dot(h, w2, preferred_element_type=jnp.float32)
        acc = hw if acc is None else jnp.maximum(acc, hw)
    out_ref[...] = acc + b2_ref[...]


@functools.lru_cache(maxsize=None)
def _make_edge(dg, dh, r):
    return _pcall(
        _edge_body,
        grid=(N // r,),
        in_specs=[
            pl.BlockSpec((K, r, dg), lambda i: (0, i, 0)),
            pl.BlockSpec((r, dg), lambda i: (i, 0)),
            pl.BlockSpec((dg, dh), lambda i: (0, 0)),
            pl.BlockSpec((1, dh), lambda i: (0, 0)),
        ],
        out_specs=pl.BlockSpec((r, dh), lambda i: (i, 0)),
        out_shape=jax.ShapeDtypeStruct((N, dh), jnp.float32),
    )


@functools.lru_cache(maxsize=None)
def _make_sc_gather(dh):
    info = plsc.get_sparse_core_info()
    nc, ns = info.num_cores, info.num_subcores
    nw = nc * ns
    tot = N * K
    per_w = tot // nw
    ch = 128
    nch = per_w // ch
    mesh = plsc.VectorSubcoreMesh(core_axis_name="c", subcore_axis_name="s")

    @functools.partial(
        pl.kernel,
        mesh=mesh,
        out_type=jax.ShapeDtypeStruct((tot, dh), jnp.float32),
        scratch_types=[
            pltpu.VMEM((ch,), jnp.int32),
            pltpu.VMEM((ch,), jnp.int32),
            pltpu.VMEM((ch, dh), jnp.float32),
            pltpu.VMEM((ch, dh), jnp.float32),
            pltpu.SemaphoreType.DMA,
            pltpu.SemaphoreType.DMA,
            pltpu.SemaphoreType.DMA,
            pltpu.SemaphoreType.DMA,
        ],
    )
    def gather(table_hbm, idx_hbm, out_hbm, idx0, idx1, rows0, rows1,
               sg0, sg1, sw0, sw1):
        wid = lax.axis_index("s") * nc + lax.axis_index("c")
        base = wid * per_w

        def body(j, carry):
            for p, (idx_v, rows_v, sg, sw) in enumerate(
                    ((idx0, rows0, sg0, sw0), (idx1, rows1, sg1, sw1))):
                off = base + (2 * j + p) * ch
                pltpu.sync_copy(idx_hbm.at[pl.ds(off, ch)], idx_v)

                @pl.when(j > 0)
                def _():
                    pltpu.make_async_copy(
                        rows_v, out_hbm.at[pl.ds(off, ch)], sw).wait()
                pltpu.make_async_copy(table_hbm.at[idx_v], rows_v, sg).start()
            for p, (idx_v, rows_v, sg, sw) in enumerate(
                    ((idx0, rows0, sg0, sw0), (idx1, rows1, sg1, sw1))):
                off = base + (2 * j + p) * ch
                pltpu.make_async_copy(table_hbm.at[idx_v], rows_v, sg).wait()
                pltpu.make_async_copy(
                    rows_v, out_hbm.at[pl.ds(off, ch)], sw).start()
            return carry

        lax.fori_loop(0, nch // 2, body, 0)
        last = base + (nch - 2) * ch
        pltpu.make_async_copy(rows0, out_hbm.at[pl.ds(last, ch)], sw0).wait()
        pltpu.make_async_copy(
            rows1, out_hbm.at[pl.ds(last + ch, ch)], sw1).wait()

    return gather


def _gather_rows(v, idx_flat, dh):
    return _make_sc_gather(dh)(v, idx_flat)


def _pool_body(x_ref, brow_ref, out_ref):
    @pl.when(pl.program_id(0) == 0)
    def _():
        out_ref[...] = jnp.full_like(out_ref, -jnp.inf)

    x = x_ref[...]
    b = brow_ref[...]
    rows = []
    for s in range(NSEG):
        rows.append(jnp.max(jnp.where(b == s, x, -jnp.inf), axis=0))
    out_ref[...] = jnp.maximum(out_ref[...], jnp.stack(rows, axis=0))


@functools.lru_cache(maxsize=None)
def _make_pool(f, r):
    return _pcall(
        _pool_body,
        grid=(N // r,),
        in_specs=[
            pl.BlockSpec((r, f), lambda i: (i, 0)),
            pl.BlockSpec((r, 1), lambda i: (i, 0)),
        ],
        out_specs=pl.BlockSpec((NSEG, f), lambda i: (0, 0)),
        out_shape=jax.ShapeDtypeStruct((NSEG, f), jnp.float32),
    )


def _head_body(p_ref, w1_ref, b1_ref, gam_ref, bet_ref, w2_ref, b2_ref, out_ref):
    h = jnp.dot(p_ref[...], w1_ref[...], preferred_element_type=jnp.float32) + b1_ref[...]
    inv = jnp.float32(1.0) / jnp.sqrt(jnp.float32(1.0 + 1e-5))
    h = h * (gam_ref[...] * inv) + bet_ref[...]
    h = jnp.maximum(h, 0.0)
    o = jnp.dot(h, w2_ref[...], preferred_element_type=jnp.float32) + b2_ref[...]
    m = jnp.max(o, axis=1, keepdims=True)
    ls = jnp.log(jnp.sum(jnp.exp(o - m), axis=1, keepdims=True))
    out_ref[...] = o - m - ls


def _make_head(fin, fmid, fout):
    return _pcall(
        _head_body,
        in_specs=[
            pl.BlockSpec((NSEG, fin), lambda: (0, 0)),
            pl.BlockSpec((fin, fmid), lambda: (0, 0)),
            pl.BlockSpec((1, fmid), lambda: (0, 0)),
            pl.BlockSpec((1, fmid), lambda: (0, 0)),
            pl.BlockSpec((1, fmid), lambda: (0, 0)),
            pl.BlockSpec((fmid, fout), lambda: (0, 0)),
            pl.BlockSpec((1, fout), lambda: (0, 0)),
        ],
        out_specs=pl.BlockSpec((NSEG, fout), lambda: (0, 0)),
        out_shape=jax.ShapeDtypeStruct((NSEG, fout), jnp.float32),
    )


def _edge_conv(x, brow, bcol, W1, b1, W2, b2, din_real, r_knn, r_edge):
    d = x.shape[1]
    dh = W2.shape[0]
    dg = max(dh, 128)
    Wv = W1[din_real:, :]
    Wu = W1[:din_real, :] - Wv
    if d != din_real:
        pad = ((0, d - din_real), (0, 0))
        Wu = jnp.pad(Wu, pad)
        Wv = jnp.pad(Wv, pad)
    b1p = b1.reshape(1, -1)
    W2p = W2
    if dg != dh:
        cpad = ((0, 0), (0, dg - dh))
        Wu = jnp.pad(Wu, cpad)
        Wv = jnp.pad(Wv, cpad)
        b1p = jnp.pad(b1p, cpad)
        W2p = jnp.pad(W2, ((0, dg - dh), (0, 0)))
    idx, u, v = _make_knn_uv(d, dg, r_knn)(
        x.T, x, brow, bcol, Wu, b1p, Wv)
    idx_flat = idx.T.reshape(-1)
    g = _gather_rows(v, idx_flat, dg).reshape(K, N, dg)
    return _make_edge(dg, dh, r_edge)(g, u, W2p, b2.reshape(1, -1))


def kernel(pos, params, batch):
    p = params
    brow = batch.reshape(N, 1).astype(jnp.int32)
    bcol = batch.reshape(1, N).astype(jnp.int32)

    x0 = jnp.pad(pos, ((0, 0), (0, 5)))
    x1 = _edge_conv(x0, brow, bcol, p['c1_W1'], p['c1_b1'], p['c1_W2'], p['c1_b2'],
                    3, 256, 256)
    x2 = _edge_conv(x1, brow, bcol, p['c2_W1'], p['c2_b1'], p['c2_W2'], p['c2_b2'],
                    64, 256, 256)
    x3 = _edge_conv(x2, brow, bcol, p['c3_W1'], p['c3_b1'], p['c3_W2'], p['c3_b2'],
                    64, 256, 256)
    x4 = _edge_conv(x3, brow, bcol, p['c4_W1'], p['c4_b1'], p['c4_W2'], p['c4_b2'],
                    128, 256, 256)

    xc = jnp.concatenate([x1, x2, x3, x4], axis=1)
    pooled = _make_pool(512, 256)(xc, brow)
    return _make_head(512, 1024, 40)(
        pooled, p['lin1_W'], p['lin1_b'].reshape(1, -1),
        p['bn1_gamma'].reshape(1, -1), p['bn1_beta'].reshape(1, -1),
        p['lin2_W'], p['lin2_b'].reshape(1, -1))
